# baseline (speedup 1.0000x reference)
import functools

import jax
import jax.numpy as jnp
from jax import lax
from jax.experimental import pallas as pl
from jax.experimental.pallas import tpu as pltpu
from jax.experimental.pallas import tpu_sc as plsc

_NC = 2
_NS = 16
_NW = _NC * _NS


@functools.lru_cache(maxsize=None)
def _make_gather(V, D, B, C, K):
    assert B % (_NW * C) == 0 and C % K == 0
    rows_per_worker = B // _NW
    num_chunks = rows_per_worker // C
    mesh = plsc.VectorSubcoreMesh(core_axis_name="c", subcore_axis_name="s")

    @functools.partial(
        pl.kernel,
        mesh=mesh,
        out_type=jax.ShapeDtypeStruct((B, D), jnp.float32),
        scratch_types=[
            pltpu.VMEM((C,), jnp.int32),
            pltpu.VMEM((C,), jnp.int32),
            pltpu.VMEM((C, D), jnp.float32),
            pltpu.VMEM((C, D), jnp.float32),
            pltpu.SemaphoreType.DMA,
            pltpu.SemaphoreType.DMA,
            pltpu.SemaphoreType.DMA,
            pltpu.SemaphoreType.DMA,
            pltpu.SemaphoreType.DMA,
            pltpu.SemaphoreType.DMA,
        ],
        compiler_params=pltpu.CompilerParams(use_tc_tiling_on_sc=False),
    )
    def k(t_hbm, idx_hbm, out_hbm,
          i0, i1, r0, r1, si0, si1, sg0, sg1, so0, so1):
        idx_v = (i0, i1)
        rows_v = (r0, r1)
        sem_i = (si0, si1)
        sem_o = (so0, so1)
        sem_g = (sg0, sg1)
        wid = lax.axis_index("s") * _NC + lax.axis_index("c")
        base = wid * rows_per_worker

        idx_d = {}
        out_d = {}
        for j in range(min(2, num_chunks)):
            idx_d[j] = pltpu.async_copy(
                idx_hbm.at[pl.ds(base + j * C, C)], idx_v[j % 2],
                sem_i[j % 2])
        for j in range(num_chunks):
            b = j % 2
            idx_d[j].wait()
            if j >= 2:
                out_d[j - 2].wait()
            S = C // K
            gd = [
                pltpu.async_copy(
                    t_hbm.at[idx_v[b].at[pl.ds(t * S, S)]],
                    rows_v[b].at[pl.ds(t * S, S)], sem_g[b])
                for t in range(K)
            ]
            for d in gd:
                d.wait()
            out_d[j] = pltpu.async_copy(
                rows_v[b], out_hbm.at[pl.ds(base + j * C, C)], sem_o[b])
            if j + 2 < num_chunks:
                idx_d[j + 2] = pltpu.async_copy(
                    idx_hbm.at[pl.ds(base + (j + 2) * C, C)], idx_v[b],
                    sem_i[b])
        for j in range(max(0, num_chunks - 2), num_chunks):
            out_d[j].wait()

    return k


def kernel(weight, indices):
    V, D = weight.shape
    B = indices.size
    idx = indices.reshape(-1).astype(jnp.int32)
    out = _make_gather(V, D, B, 1600, 4)(weight, idx)
    return out.reshape(indices.shape + (D,))
